# baseline (speedup 1.0000x reference)
import functools

import jax
import jax.numpy as jnp
from jax import lax
from jax.experimental import pallas as pl
from jax.experimental.pallas import tpu as pltpu
from jax.experimental.pallas import tpu_sc as plsc

_NC = 2
_NS = 16
_NW = _NC * _NS
_L = 16


def _sc_gather_kernel(nunit, nreal, p2, topk, w2, c_kv, n, ridx_hbm, w_hbm,
                      kv_hbm, out_hbm, dump_hbm, idx_v, w_v, buf0, buf1,
                      tbuf0, tbuf1, gsem0, gsem1, wsem0, wsem1):
    wid = lax.axis_index("s") * _NC + lax.axis_index("c")
    bufs = (buf0, buf1)
    tbufs = (tbuf0, tbuf1)
    gsems = (gsem0, gsem1)
    wsems = (wsem0, wsem1)
    wof = list(range(0, w2 - 8, 8))
    wss = [8] * len(wof)
    wof.append(len(wof) * 8)
    wss.append(w2 - wof[-1])
    nchk = len(wof)

    def unit_body(q, _):
        u = wid + _NW * q
        i = u // topk
        t = u - topk * i
        ok = u < nreal
        pltpu.sync_copy(ridx_hbm.at[u], idx_v)
        pltpu.sync_copy(w_hbm.at[u], w_v)
        iv = idx_v[0]
        wv = w_v[0]
        wsp = [jnp.full((_L,), wv[b], jnp.float32) for b in range(n)]

        def start_gather(p):
            w0, ws = wof[p], wss[p]
            buf = bufs[p % 2]
            return [
                pltpu.async_copy(
                    kv_hbm.at[b, iv[b], pl.ds(w0, ws), :],
                    buf.at[b, pl.ds(0, ws), :], gsems[p % 2])
                for b in range(n)
            ]

        def scale(p):
            buf, tbuf = bufs[p % 2], tbufs[p % 2]

            def mul_body(s, _):
                for b in range(n):
                    for uc in range(c_kv // _L):
                        sl = pl.ds(uc * _L, _L)
                        tbuf[s, b, sl] = buf[b, s, sl] * wsp[b]
                return 0

            lax.fori_loop(0, wss[p], mul_body, 0)

        def start_write(p):
            w0, ws = wof[p], wss[p]
            tbuf, wsem = tbufs[p % 2], wsems[p % 2]
            src = tbuf.at[pl.ds(0, ws)]

            @pl.when(ok)
            def _():
                pltpu.async_copy(src, out_hbm.at[i, t, pl.ds(w0, ws)], wsem)

            @pl.when(jnp.logical_not(ok))
            def _():
                pltpu.async_copy(src, dump_hbm.at[pl.ds(0, ws)], wsem)

            return pltpu.make_async_copy(src, dump_hbm.at[pl.ds(0, ws)],
                                         wsem)

        gathers = {0: start_gather(0)}
        writes = {}
        for p in range(nchk):
            for cp in gathers.pop(p):
                cp.wait()
            if p + 1 < nchk:
                gathers[p + 1] = start_gather(p + 1)
            if p >= 2:
                writes.pop(p - 2).wait()
            scale(p)
            writes[p] = start_write(p)
        for p in sorted(writes):
            writes.pop(p).wait()
        return 0

    lax.fori_loop(0, nunit, unit_body, 0)


def kernel(r_idx, r_weight, kv):
    n, p2, w2, c_kv = kv.shape
    topk = r_idx.shape[-1]
    nreal = p2 * topk
    nunit = -(-nreal // _NW)
    upad = nunit * _NW - nreal

    idx_t = jnp.transpose(r_idx, (1, 2, 0)).reshape(nreal, 1, n)
    w_t = jnp.transpose(r_weight, (1, 2, 0)).reshape(nreal, 1, n)
    idx_p = jnp.pad(idx_t, ((0, upad), (0, 0), (0, _L - n)))
    w_p = jnp.pad(w_t, ((0, upad), (0, 0), (0, _L - n)))

    wmax = w2 - 8 * (w2 // 8 - 1)

    mesh = plsc.VectorSubcoreMesh(core_axis_name="c", subcore_axis_name="s")
    body = functools.partial(_sc_gather_kernel, nunit, nreal, p2, topk, w2,
                             c_kv, n)
    out, _ = pl.kernel(
        body,
        out_type=(
            jax.ShapeDtypeStruct((p2, topk, w2, n, c_kv), jnp.float32),
            jax.ShapeDtypeStruct((wmax, n, c_kv), jnp.float32),
        ),
        mesh=mesh,
        compiler_params=pltpu.CompilerParams(use_tc_tiling_on_sc=True),
        scratch_types=[
            pltpu.VMEM((1, _L), jnp.int32),
            pltpu.VMEM((1, _L), jnp.float32),
            pltpu.VMEM((n, wmax, c_kv), jnp.float32),
            pltpu.VMEM((n, wmax, c_kv), jnp.float32),
            pltpu.VMEM((wmax, n, c_kv), jnp.float32),
            pltpu.VMEM((wmax, n, c_kv), jnp.float32),
            pltpu.SemaphoreType.DMA,
            pltpu.SemaphoreType.DMA,
            pltpu.SemaphoreType.DMA,
            pltpu.SemaphoreType.DMA,
        ],
    )(idx_p, w_p, kv)
    return jnp.transpose(out, (3, 0, 1, 2, 4))


# speedup vs baseline: 1.0291x; 1.0291x over previous
import functools

import jax
import jax.numpy as jnp
from jax import lax
from jax.experimental import pallas as pl
from jax.experimental.pallas import tpu as pltpu
from jax.experimental.pallas import tpu_sc as plsc

_NC = 2
_NS = 16
_NW = _NC * _NS
_L = 16


def _sc_gather_kernel(nunit, nreal, p2, topk, w2, c_kv, n, idx_hbm, wt_hbm,
                      kv_hbm, out_hbm, dump_hbm, idx_v, wt_v, buf0, buf1,
                      tbuf0, tbuf1, gsem0, gsem1, wsem0, wsem1):
    wid = lax.axis_index("s") * _NC + lax.axis_index("c")
    bufs = (buf0, buf1)
    tbufs = (tbuf0, tbuf1)
    gsems = (gsem0, gsem1)
    wsems = (wsem0, wsem1)
    wof = list(range(0, w2 - 8, 8))
    wss = [8] * len(wof)
    wof.append(len(wof) * 8)
    wss.append(w2 - wof[-1])
    nchk = len(wof)

    pltpu.sync_copy(idx_hbm.at[wid], idx_v)
    pltpu.sync_copy(wt_hbm.at[wid], wt_v)

    def unit_body(q, _):
        u = wid + _NW * q
        i = u // topk
        t = u - topk * i
        ok = u < nreal
        iv = idx_v[q]
        wv = wt_v[q]
        wsp = [jnp.full((_L,), wv[b], jnp.float32) for b in range(n)]

        def start_gather(p):
            w0, ws = wof[p], wss[p]
            buf = bufs[p % 2]
            return [
                pltpu.async_copy(
                    kv_hbm.at[b, iv[b], pl.ds(w0, ws), :],
                    buf.at[b, pl.ds(0, ws), :], gsems[p % 2])
                for b in range(n)
            ]

        def scale(p):
            buf, tbuf = bufs[p % 2], tbufs[p % 2]

            def mul_body(s, _):
                for b in range(n):
                    for uc in range(c_kv // _L):
                        sl = pl.ds(uc * _L, _L)
                        tbuf[s, b, sl] = buf[b, s, sl] * wsp[b]
                return 0

            lax.fori_loop(0, wss[p], mul_body, 0)

        def start_write(p):
            w0, ws = wof[p], wss[p]
            tbuf, wsem = tbufs[p % 2], wsems[p % 2]
            src = tbuf.at[pl.ds(0, ws)]

            @pl.when(ok)
            def _():
                pltpu.async_copy(src, out_hbm.at[i, t, pl.ds(w0, ws)], wsem)

            @pl.when(jnp.logical_not(ok))
            def _():
                pltpu.async_copy(src, dump_hbm.at[pl.ds(0, ws)], wsem)

            return pltpu.make_async_copy(src, dump_hbm.at[pl.ds(0, ws)],
                                         wsem)

        gathers = {0: start_gather(0), 1: start_gather(1)}
        writes = {}
        for p in range(nchk):
            for cp in gathers.pop(p):
                cp.wait()
            if p >= 2:
                writes.pop(p - 2).wait()
            scale(p)
            writes[p] = start_write(p)
            if p + 2 < nchk:
                gathers[p + 2] = start_gather(p + 2)
        for p in sorted(writes):
            writes.pop(p).wait()
        return 0

    lax.fori_loop(0, nunit, unit_body, 0)


def kernel(r_idx, r_weight, kv):
    n, p2, w2, c_kv = kv.shape
    topk = r_idx.shape[-1]
    nreal = p2 * topk
    nunit = -(-nreal // _NW)
    upad = nunit * _NW - nreal

    def worker_major(a):
        a = jnp.transpose(a, (1, 2, 0)).reshape(nreal, n)
        a = jnp.pad(a, ((0, upad), (0, _L - n)))
        return a.reshape(nunit, _NW, _L).transpose(1, 0, 2)

    idx_w = worker_major(r_idx)
    wt_w = worker_major(r_weight)

    wmax = w2 - 8 * (w2 // 8 - 1)

    mesh = plsc.VectorSubcoreMesh(core_axis_name="c", subcore_axis_name="s")
    body = functools.partial(_sc_gather_kernel, nunit, nreal, p2, topk, w2,
                             c_kv, n)
    out, _ = pl.kernel(
        body,
        out_type=(
            jax.ShapeDtypeStruct((p2, topk, w2, n, c_kv), jnp.float32),
            jax.ShapeDtypeStruct((wmax, n, c_kv), jnp.float32),
        ),
        mesh=mesh,
        compiler_params=pltpu.CompilerParams(use_tc_tiling_on_sc=True),
        scratch_types=[
            pltpu.VMEM((nunit, _L), jnp.int32),
            pltpu.VMEM((nunit, _L), jnp.float32),
            pltpu.VMEM((n, wmax, c_kv), jnp.float32),
            pltpu.VMEM((n, wmax, c_kv), jnp.float32),
            pltpu.VMEM((wmax, n, c_kv), jnp.float32),
            pltpu.VMEM((wmax, n, c_kv), jnp.float32),
            pltpu.SemaphoreType.DMA,
            pltpu.SemaphoreType.DMA,
            pltpu.SemaphoreType.DMA,
            pltpu.SemaphoreType.DMA,
        ],
    )(idx_w, wt_w, kv)
    return jnp.transpose(out, (3, 0, 1, 2, 4))


# speedup vs baseline: 1.2968x; 1.2600x over previous
import functools

import jax
import jax.numpy as jnp
from jax import lax
from jax.experimental import pallas as pl
from jax.experimental.pallas import tpu as pltpu
from jax.experimental.pallas import tpu_sc as plsc

_NC = 2
_NS = 16
_NW = _NC * _NS
_L = 16
_CH = 4


def _sc_gather_kernel(nch, nreal, p2, topk, w2, c_kv, gidx_hbm, w_hbm,
                      kv_hbm, out_hbm, dump_hbm, idx_v, w_v, buf0, buf1,
                      buf2, gsem0, gsem1, gsem2, wsem0, wsem1, wsem2):
    wid = lax.axis_index("s") * _NC + lax.axis_index("c")
    pltpu.sync_copy(gidx_hbm.at[wid], idx_v)
    pltpu.sync_copy(w_hbm.at[wid], w_v)

    bufs = (buf0, buf1, buf2)
    gsems = (gsem0, gsem1, gsem2)
    wsems = (wsem0, wsem1, wsem2)
    nchw = topk // _CH

    def start_gather(q):
        return pltpu.async_copy(
            kv_hbm.at[idx_v.at[q]], bufs[q % 3], gsems[q % 3])

    def scale(q):
        buf = bufs[q % 3]
        wsp = [w_v[q, j] for j in range(_CH)]

        def mul_body(s, _):
            for j in range(_CH):
                for u in range(c_kv // _L):
                    sl = pl.ds(u * _L, _L)
                    buf[j, s, sl] = buf[j, s, sl] * wsp[j]
            return 0

        lax.fori_loop(0, w2, mul_body, 0)

    def start_writes(q):
        buf = bufs[q % 3]
        wsem = wsems[q % 3]
        h = wid + _NW * q
        r = h // nchw
        c = h - nchw * r
        b = r // p2
        i = r - p2 * b
        cps = []
        for j in range(_CH):
            dst = out_hbm.at[i, c * _CH + j, :, b, :]
            if q == nch - 1:
                ok = h < nreal

                @pl.when(ok)
                def _(dst=dst, j=j):
                    pltpu.async_copy(buf.at[j], dst, wsem)

                @pl.when(jnp.logical_not(ok))
                def _(j=j):
                    pltpu.async_copy(buf.at[j], dump_hbm.at[j], wsem)

                cps.append(pltpu.make_async_copy(buf.at[j], dump_hbm.at[j],
                                                 wsem))
            else:
                cps.append(pltpu.async_copy(buf.at[j], dst, wsem))
        return cps

    writes = {}
    gathers = {0: start_gather(0), 1: start_gather(1)}
    for q in range(nch):
        gathers.pop(q).wait()
        scale(q)
        writes[q] = start_writes(q)
        if q + 2 < nch:
            if q >= 1:
                for cp in writes.pop(q - 1):
                    cp.wait()
            gathers[q + 2] = start_gather(q + 2)
    for q in sorted(writes):
        for cp in writes.pop(q):
            cp.wait()


def kernel(r_idx, r_weight, kv):
    n, p2, w2, c_kv = kv.shape
    topk = r_idx.shape[-1]
    R = n * p2
    kv_tab = kv.reshape(R, w2, c_kv)
    nchunks = R * topk // _CH
    nch = -(-nchunks // _NW)

    gidx = (jnp.arange(n, dtype=jnp.int32)[:, None, None] * p2
            + r_idx).reshape(nchunks, _CH)
    wflat = r_weight.reshape(nchunks, _CH)
    padc = nch * _NW - nchunks
    gidx_p = jnp.pad(gidx, ((0, padc), (0, 0))).reshape(nch, _NW, _CH)
    gidx_w = jnp.transpose(gidx_p, (1, 0, 2))
    qpad = 8 * (-(-nch // 8)) - nch
    gidx_w = jnp.pad(gidx_w, ((0, 0), (0, qpad), (0, 0)))
    w_p = jnp.pad(wflat, ((0, padc), (0, 0))).reshape(nch, _NW, _CH)
    w_w = jnp.pad(jnp.transpose(w_p, (1, 0, 2)), ((0, 0), (0, qpad), (0, 0)))
    w_w = jnp.broadcast_to(w_w[:, :, :, None], (_NW, nch + qpad, _CH, _L))

    mesh = plsc.VectorSubcoreMesh(core_axis_name="c", subcore_axis_name="s")
    body = functools.partial(_sc_gather_kernel, nch, nchunks, p2, topk, w2,
                             c_kv)
    out, _ = pl.kernel(
        body,
        out_type=(
            jax.ShapeDtypeStruct((p2, topk, w2, n, c_kv), jnp.float32),
            jax.ShapeDtypeStruct((_CH, w2, c_kv), jnp.float32),
        ),
        mesh=mesh,
        compiler_params=pltpu.CompilerParams(use_tc_tiling_on_sc=False),
        scratch_types=[
            pltpu.VMEM((nch + qpad, _CH), jnp.int32),
            pltpu.VMEM((nch + qpad, _CH, _L), jnp.float32),
            pltpu.VMEM((_CH, w2, c_kv), jnp.float32),
            pltpu.VMEM((_CH, w2, c_kv), jnp.float32),
            pltpu.VMEM((_CH, w2, c_kv), jnp.float32),
            pltpu.SemaphoreType.DMA,
            pltpu.SemaphoreType.DMA,
            pltpu.SemaphoreType.DMA,
            pltpu.SemaphoreType.DMA,
            pltpu.SemaphoreType.DMA,
            pltpu.SemaphoreType.DMA,
        ],
    )(gidx_w, w_w, kv_tab)
    return jnp.transpose(out, (3, 0, 1, 2, 4))
